# baseline (speedup 1.0000x reference)
import functools

import jax
import jax.numpy as jnp
from jax import lax
from jax.experimental import pallas as pl
from jax.experimental.pallas import tpu as pltpu
from jax.experimental.pallas import tpu_sc as plsc

NENTITY = 1000000
NRELATION = 10000
HIDDEN = 32
TPDIM = 32
NTP = 4
BATCH = 16384
GAMMA = 12.0
ALPHA = 0.5

_ED = HIDDEN + TPDIM
_NC = 2
_NS = 16
_NW = _NC * _NS
_BPW = BATCH // _NW
_CHUNK = 128
_NCHUNK = _BPW // _CHUNK
_L = 16
_NGROUP = _BPW // _L


def _kge_body(idx, ent, tpr,
              out,
              hidx_v, ridx_v, tidx_v,
              h_v, rel_v, t_v,
              tpr_v, out_v, sem):
    wid = lax.axis_index("s") * _NC + lax.axis_index("c")

    pltpu.sync_copy(idx.at[0, wid], hidx_v)
    pltpu.sync_copy(idx.at[1, wid], ridx_v)
    pltpu.sync_copy(idx.at[2, wid], tidx_v)
    pltpu.sync_copy(tpr, tpr_v)

    copies = []
    for j in range(_NCHUNK):
        rows = pl.ds(j * _CHUNK, _CHUNK)
        copies.append(pltpu.async_copy(ent.at[hidx_v.at[j]], h_v.at[rows, :], sem))
        copies.append(pltpu.async_copy(ent.at[ridx_v.at[j]], rel_v.at[rows, :], sem))
        copies.append(pltpu.async_copy(ent.at[tidx_v.at[j]], t_v.at[rows, :], sem))
    for c in copies:
        c.wait()

    lanes = lax.iota(jnp.int32, _L)
    tpr_r = [[tpr_v[pl.ds(k * TPDIM + half * _L, _L)] for half in range(2)]
             for k in range(NTP)]

    def group(g, carry):
        scores = jnp.zeros((_L,), jnp.float32)
        for el in range(_L):
            e = g * _L + el
            h = [h_v[e, pl.ds(q * _L, _L)] for q in range(4)]
            t = [t_v[e, pl.ds(q * _L, _L)] for q in range(4)]
            r = [rel_v[e, pl.ds(q * _L, _L)] for q in range(4)]
            svec = (jnp.abs(h[0] + r[0] - t[0]) + jnp.abs(h[1] + r[1] - t[1])
                    + jnp.abs(h[2] + r[2] - t[2]) + jnp.abs(h[3] + r[3] - t[3]))
            s = jnp.sum(svec)
            b0 = h[2] - t[2]
            b1 = h[3] - t[3]
            tps = [jnp.sum(jnp.abs(b0 + tpr_r[k][0]) + jnp.abs(b1 + tpr_r[k][1]))
                   for k in range(NTP)]
            tp_min = jnp.minimum(jnp.minimum(tps[0], tps[1]),
                                 jnp.minimum(tps[2], tps[3]))
            score = GAMMA - s - ALPHA * tp_min
            scores = jnp.where(lanes == el, score, scores)
        out_v[pl.ds(g * _L, _L)] = scores
        return carry

    lax.fori_loop(0, _NGROUP, group, 0)

    pltpu.sync_copy(out_v, out.at[pl.ds(wid * _BPW, _BPW)])


@jax.jit
def _kge(idx, ent, tpr_flat):
    mesh = plsc.VectorSubcoreMesh(core_axis_name="c", subcore_axis_name="s")
    f = functools.partial(
        pl.kernel, mesh=mesh,
        compiler_params=pltpu.CompilerParams(
            needs_layout_passes=False, use_tc_tiling_on_sc=False),
        out_type=jax.ShapeDtypeStruct((BATCH,), jnp.float32),
        scratch_types=[
            pltpu.VMEM((_NCHUNK, _CHUNK), jnp.int32),
            pltpu.VMEM((_NCHUNK, _CHUNK), jnp.int32),
            pltpu.VMEM((_NCHUNK, _CHUNK), jnp.int32),
            pltpu.VMEM((_BPW, _ED), jnp.float32),
            pltpu.VMEM((_BPW, _ED), jnp.float32),
            pltpu.VMEM((_BPW, _ED), jnp.float32),
            pltpu.VMEM((NTP * TPDIM,), jnp.float32),
            pltpu.VMEM((_BPW,), jnp.float32),
            pltpu.SemaphoreType.DMA,
        ],
    )(_kge_body)
    return f(idx, ent, tpr_flat)


def kernel(sample, ent_embed_struct, ent_embed_tp, rel_emb, tp_rel):
    shift = jnp.array([[0], [1], [0]], dtype=sample.dtype)
    idx = (2 * sample.T + shift).reshape(3, _NW, _NCHUNK, _CHUNK)
    eye = jnp.eye(HIDDEN, dtype=jnp.float32)
    z = jnp.zeros((HIDDEN, HIDDEN), dtype=jnp.float32)
    sel_s = jnp.concatenate([eye, z, z, z], axis=1)
    sel_t = jnp.concatenate([z, eye, z, z], axis=1)
    sel_r = jnp.concatenate([jnp.zeros((_ED, _ED), jnp.float32),
                             jnp.eye(_ED, dtype=jnp.float32)], axis=1)
    dot = functools.partial(jnp.dot, precision=jax.lax.Precision.HIGHEST)
    pack = (dot(ent_embed_struct[:NRELATION], sel_s)
            + dot(ent_embed_tp[:NRELATION], sel_t)
            + dot(rel_emb, sel_r))
    pack2 = pack.reshape(2 * NRELATION, _ED)
    score = _kge(idx, pack2, tp_rel.reshape(NTP * TPDIM))
    return score.reshape(BATCH, 1)


# speedup vs baseline: 1.6431x; 1.6431x over previous
import functools

import jax
import jax.numpy as jnp
from jax import lax
from jax.experimental import pallas as pl
from jax.experimental.pallas import tpu as pltpu
from jax.experimental.pallas import tpu_sc as plsc

NENTITY = 1000000
NRELATION = 10000
HIDDEN = 32
TPDIM = 32
NTP = 4
BATCH = 16384
GAMMA = 12.0
ALPHA = 0.5

_ED = HIDDEN + TPDIM
_NC = 2
_NS = 16
_NW = _NC * _NS
_BPW = BATCH // _NW
_CHUNK = 128
_NCHUNK = _BPW // _CHUNK
_L = 16
_NGROUP = _BPW // _L


def _kge_body(idx, ent, tpr,
              out,
              hidx_v, ridx_v, tidx_v,
              h_v, rel_v, t_v,
              tpr_v, out_v, sem):
    wid = lax.axis_index("s") * _NC + lax.axis_index("c")

    pltpu.sync_copy(idx.at[0, wid], hidx_v)
    pltpu.sync_copy(idx.at[1, wid], ridx_v)
    pltpu.sync_copy(idx.at[2, wid], tidx_v)
    pltpu.sync_copy(tpr, tpr_v)

    copies = []
    for j in range(_NCHUNK):
        rows = pl.ds(j * _CHUNK, _CHUNK)
        copies.append(pltpu.async_copy(ent.at[hidx_v.at[j]], h_v.at[rows, :], sem))
        copies.append(pltpu.async_copy(ent.at[ridx_v.at[j]], rel_v.at[rows, :], sem))
        copies.append(pltpu.async_copy(ent.at[tidx_v.at[j]], t_v.at[rows, :], sem))
    for c in copies:
        c.wait()

    lanes = lax.iota(jnp.int32, _L)
    tpr_r = [[tpr_v[pl.ds(k * TPDIM + half * _L, _L)] for half in range(2)]
             for k in range(NTP)]

    def group(g, carry):
        scores = jnp.zeros((_L,), jnp.float32)
        for el in range(_L):
            e = g * _L + el
            h = [h_v[e, pl.ds(q * _L, _L)] for q in range(4)]
            t = [t_v[e, pl.ds(q * _L, _L)] for q in range(4)]
            r = [rel_v[e, pl.ds(q * _L, _L)] for q in range(4)]
            svec = (jnp.abs(h[0] + r[0] - t[0]) + jnp.abs(h[1] + r[1] - t[1])
                    + jnp.abs(h[2] + r[2] - t[2]) + jnp.abs(h[3] + r[3] - t[3]))
            s = jnp.sum(svec)
            b0 = h[2] - t[2]
            b1 = h[3] - t[3]
            tps = [jnp.sum(jnp.abs(b0 + tpr_r[k][0]) + jnp.abs(b1 + tpr_r[k][1]))
                   for k in range(NTP)]
            tp_min = jnp.minimum(jnp.minimum(tps[0], tps[1]),
                                 jnp.minimum(tps[2], tps[3]))
            score = GAMMA - s - ALPHA * tp_min
            scores = jnp.where(lanes == el, score, scores)
        out_v[pl.ds(g * _L, _L)] = scores
        return carry

    lax.fori_loop(0, _NGROUP, group, 0)

    pltpu.sync_copy(out_v, out.at[pl.ds(wid * _BPW, _BPW)])


@jax.jit
def _kge(idx, ent, tpr_flat):
    mesh = plsc.VectorSubcoreMesh(core_axis_name="c", subcore_axis_name="s")
    f = functools.partial(
        pl.kernel, mesh=mesh,
        compiler_params=pltpu.CompilerParams(
            needs_layout_passes=False, use_tc_tiling_on_sc=False),
        out_type=jax.ShapeDtypeStruct((BATCH,), jnp.float32),
        scratch_types=[
            pltpu.VMEM((_NCHUNK, _CHUNK), jnp.int32),
            pltpu.VMEM((_NCHUNK, _CHUNK), jnp.int32),
            pltpu.VMEM((_NCHUNK, _CHUNK), jnp.int32),
            pltpu.VMEM((_BPW, _ED), jnp.float32),
            pltpu.VMEM((_BPW, _ED), jnp.float32),
            pltpu.VMEM((_BPW, _ED), jnp.float32),
            pltpu.VMEM((NTP * TPDIM,), jnp.float32),
            pltpu.VMEM((_BPW,), jnp.float32),
            pltpu.SemaphoreType.DMA,
        ],
    )(_kge_body)
    return f(idx, ent, tpr_flat)


def kernel(sample, ent_embed_struct, ent_embed_tp, rel_emb, tp_rel):
    shift = jnp.array([[0], [1], [0]], dtype=sample.dtype)
    idx = (2 * sample.T + shift).reshape(3, _NW, _NCHUNK, _CHUNK)
    pack = (jnp.pad(ent_embed_struct[:NRELATION], ((0, 0), (0, 96)))
            + jnp.pad(ent_embed_tp[:NRELATION], ((0, 0), (32, 64)))
            + jnp.pad(rel_emb, ((0, 0), (64, 0))))
    pack2 = pack.reshape(2 * NRELATION, _ED)
    score = _kge(idx, pack2, tp_rel.reshape(NTP * TPDIM))
    return score.reshape(BATCH, 1)
